# baseline (speedup 1.0000x reference)
import functools

import jax
import jax.numpy as jnp
from jax import lax
from jax.experimental import pallas as pl
from jax.experimental.pallas import tpu as pltpu
from jax.experimental.pallas import tpu_sc as plsc

B = 16384
VOCAB1 = 1000001
DIM = 32
ODIM = 2 * DIM + 1
NBUCKETS = 1000
L = 16

_NC = 2
_NS = 16
_NW = _NC * _NS
_BPW = B // _NW

_STEPS = (512, 256, 128, 64, 32, 16, 8, 4, 2, 1)


def _body(uid_hbm, ts_hbm, utab_hbm, ttab_hbm, bkt_hbm, mean_hbm, scale_hbm,
          out_hbm, idx_v, widx_v, uflat_v, ts_v, tsidx_v, trows_v, out_v,
          bkt_v, ms_v, sem_u, sem_t):
    wid = lax.axis_index("s") * _NC + lax.axis_index("c")
    base = wid * _BPW

    pltpu.sync_copy(uid_hbm.at[pl.ds(base, _BPW)], idx_v)
    pltpu.sync_copy(ts_hbm.at[pl.ds(base, _BPW)], ts_v)
    pltpu.sync_copy(bkt_hbm, bkt_v)
    pltpu.sync_copy(mean_hbm, ms_v.at[pl.ds(0, L)])
    pltpu.sync_copy(scale_hbm, ms_v.at[pl.ds(L, L)])

    lane = lax.iota(jnp.int32, L)
    lane32 = DIM * lane

    def expand(i, carry):
        off = pl.multiple_of(i * L, L)
        uvec = idx_v[pl.ds(off, L)]
        pos0 = (off * DIM) + lane32
        for f in range(DIM):
            plsc.store_scatter(widx_v, [pos0 + f], uvec + (f * VOCAB1))
        return carry

    lax.fori_loop(0, _BPW // L, expand, 0)

    cp_u = pltpu.make_async_copy(utab_hbm.at[widx_v], uflat_v, sem_u)
    cp_u.start()

    mean = ms_v[pl.ds(0, L)]
    scale = ms_v[pl.ds(L, L)]

    def bucketize(i, carry):
        off = pl.multiple_of(i * L, L)
        t = ts_v[pl.ds(off, L)]
        pos = jnp.zeros((L,), jnp.int32)
        for step in _STEPS:
            cand = pos + step
            safe = jnp.minimum(cand - 1, NBUCKETS - 1)
            bv = plsc.load_gather(bkt_v, [safe])
            take = jnp.logical_and(cand <= NBUCKETS, bv < t)
            pos = jnp.where(take, cand, pos)
        tsidx_v[pl.ds(off, L)] = pos
        rows = off + lane
        plsc.store_scatter(out_v, [rows, jnp.full((L,), 2 * DIM, jnp.int32)],
                           (t - mean) * scale)
        return carry

    lax.fori_loop(0, _BPW // L, bucketize, 0)

    cp_t = pltpu.make_async_copy(ttab_hbm.at[tsidx_v], trows_v, sem_t)
    cp_t.start()
    cp_u.wait()
    cp_t.wait()

    def assemble(r, carry):
        u0 = pl.multiple_of(r * DIM, L)
        out_v[r, pl.ds(0, L)] = uflat_v[pl.ds(u0, L)]
        out_v[r, pl.ds(L, L)] = uflat_v[pl.ds(u0 + L, L)]
        out_v[r, pl.ds(2 * L, L)] = trows_v[r, pl.ds(0, L)]
        out_v[r, pl.ds(3 * L, L)] = trows_v[r, pl.ds(L, L)]
        return carry

    lax.fori_loop(0, _BPW, assemble, 0)

    pltpu.sync_copy(out_v, out_hbm.at[pl.ds(base, _BPW)])


@jax.jit
def _run(user_id, timestamp, utab_flat, ts_table, buckets, mean16, scale16):
    mesh = plsc.VectorSubcoreMesh(core_axis_name="c", subcore_axis_name="s")
    f = functools.partial(
        pl.kernel,
        mesh=mesh,
        out_type=jax.ShapeDtypeStruct((B, ODIM), jnp.float32),
        scratch_types=[
            pltpu.VMEM((_BPW,), jnp.int32),
            pltpu.VMEM((_BPW * DIM,), jnp.int32),
            pltpu.VMEM((_BPW * DIM,), jnp.float32),
            pltpu.VMEM((_BPW,), jnp.float32),
            pltpu.VMEM((_BPW,), jnp.int32),
            pltpu.VMEM((_BPW, DIM), jnp.float32),
            pltpu.VMEM((_BPW, ODIM), jnp.float32),
            pltpu.VMEM((NBUCKETS,), jnp.float32),
            pltpu.VMEM((2 * L,), jnp.float32),
            pltpu.SemaphoreType.DMA,
            pltpu.SemaphoreType.DMA,
        ],
        compiler_params=pltpu.CompilerParams(use_tc_tiling_on_sc=False,
                                             needs_layout_passes=False),
    )(_body)
    return f(user_id, timestamp, utab_flat, ts_table, buckets, mean16,
             scale16)


def kernel(user_id, timestamp, user_table, ts_table, buckets, norm_mean,
           norm_var):
    scale = lax.rsqrt(norm_var[0] + 1e-6)
    mean16 = jnp.broadcast_to(norm_mean[0], (L,))
    scale16 = jnp.broadcast_to(scale, (L,))
    utab_flat = user_table.T.reshape(-1)
    return _run(user_id.astype(jnp.int32), timestamp, utab_flat, ts_table,
                buckets, mean16, scale16)


# speedup vs baseline: 14.1263x; 14.1263x over previous
import functools

import jax
import jax.numpy as jnp
from jax import lax
from jax.experimental import pallas as pl
from jax.experimental.pallas import tpu as pltpu
from jax.experimental.pallas import tpu_sc as plsc

B = 16384
VOCAB1 = 1000001
DIM = 32
ODIM = 2 * DIM + 1
NBUCKETS = 1000
TSROWS = 1024
L = 16

_NC = 2
_NS = 16
_NW = _NC * _NS
_BPW = B // _NW
_NBUF = 4

_STEPS = (512, 256, 128, 64, 32, 16, 8, 4, 2, 1)


def _body(uid_hbm, ts_hbm, utab_hbm, ttab_hbm, bkt_hbm, mean_hbm, scale_hbm,
          out_hbm, idx_v, ts_v, tsj_v, ring_v, ttab_v, out_v, bkt_v, ms_v,
          sems, sem_t):
    wid = lax.axis_index("s") * _NC + lax.axis_index("c")
    base = wid * _BPW
    lane = lax.iota(jnp.int32, L)

    pltpu.sync_copy(uid_hbm.at[pl.ds(base, _BPW)], idx_v.at[pl.ds(0, _BPW)])
    pltpu.sync_copy(ts_hbm.at[pl.ds(base, _BPW)], ts_v)
    pltpu.sync_copy(bkt_hbm, bkt_v)
    pltpu.sync_copy(mean_hbm, ms_v.at[pl.ds(0, L)])
    pltpu.sync_copy(scale_hbm, ms_v.at[pl.ds(L, L)])
    cp_tt = pltpu.make_async_copy(ttab_hbm, ttab_v, sem_t)
    cp_tt.start()

    def _fetch(r, slot):
        uvec = idx_v[pl.ds(r, L)]
        b = uvec[0] >> 7
        return pltpu.make_async_copy(
            utab_hbm.at[:, pl.ds(b * 128, 128)], ring_v.at[slot],
            sems.at[slot])

    for s in range(_NBUF):
        _fetch(s, s).start()

    mean = ms_v[pl.ds(0, L)]
    scale = ms_v[pl.ds(L, L)]

    def bucketize(i, carry):
        off = pl.multiple_of(i * L, L)
        t = ts_v[pl.ds(off, L)]
        pos = jnp.zeros((L,), jnp.int32)
        for step in _STEPS:
            cand = pos + step
            safe = jnp.minimum(cand - 1, NBUCKETS - 1)
            bv = plsc.load_gather(bkt_v, [safe])
            take = jnp.logical_and(cand <= NBUCKETS, bv < t)
            pos = jnp.where(take, cand, pos)
        tsj_v[pl.ds(off, L)] = pos
        rows = off + lane
        plsc.store_scatter(out_v, [rows, jnp.full((L,), 2 * DIM, jnp.int32)],
                           (t - mean) * scale)
        return carry

    lax.fori_loop(0, _BPW // L, bucketize, 0)

    def ublock(g, carry):
        r0 = pl.multiple_of(g * _NBUF, _NBUF)
        for s in range(_NBUF):
            r = r0 + s
            _fetch(r, s).wait()
            uvec = idx_v[pl.ds(r, L)]
            c = jnp.full((L,), uvec[0] & 127, jnp.int32)
            out_v[r, pl.ds(0, L)] = plsc.load_gather(ring_v.at[s], [lane, c])
            out_v[r, pl.ds(L, L)] = plsc.load_gather(ring_v.at[s],
                                                     [lane + L, c])

            @pl.when(r + _NBUF < _BPW)
            def _():
                _fetch(r + _NBUF, s).start()
        return carry

    lax.fori_loop(0, _BPW // _NBUF, ublock, 0)

    cp_tt.wait()

    def tsblock(i, carry):
        off = pl.multiple_of(i * L, L)
        jvec = tsj_v[pl.ds(off, L)]
        rows = off + lane
        for f in range(DIM):
            vals = plsc.load_gather(ttab_v, [jnp.full((L,), f, jnp.int32),
                                             jvec])
            plsc.store_scatter(out_v, [rows,
                                       jnp.full((L,), DIM + f, jnp.int32)],
                               vals)
        return carry

    lax.fori_loop(0, _BPW // L, tsblock, 0)

    pltpu.sync_copy(out_v, out_hbm.at[pl.ds(base, _BPW)])


@jax.jit
def _run(user_id, timestamp, utab_t, ttab_p, buckets, mean16, scale16):
    mesh = plsc.VectorSubcoreMesh(core_axis_name="c", subcore_axis_name="s")
    f = functools.partial(
        pl.kernel,
        mesh=mesh,
        out_type=jax.ShapeDtypeStruct((B, ODIM), jnp.float32),
        scratch_types=[
            pltpu.VMEM((_BPW + L,), jnp.int32),
            pltpu.VMEM((_BPW,), jnp.float32),
            pltpu.VMEM((_BPW,), jnp.int32),
            pltpu.VMEM((_NBUF, DIM, 128), jnp.float32),
            pltpu.VMEM((DIM, TSROWS), jnp.float32),
            pltpu.VMEM((_BPW, ODIM), jnp.float32),
            pltpu.VMEM((NBUCKETS,), jnp.float32),
            pltpu.VMEM((2 * L,), jnp.float32),
            pltpu.SemaphoreType.DMA((_NBUF,)),
            pltpu.SemaphoreType.DMA,
        ],
        compiler_params=pltpu.CompilerParams(use_tc_tiling_on_sc=True,
                                             needs_layout_passes=False,
                                             disable_bounds_checks=True),
    )(_body)
    return f(user_id, timestamp, utab_t, ttab_p, buckets, mean16, scale16)


def kernel(user_id, timestamp, user_table, ts_table, buckets, norm_mean,
           norm_var):
    scale = lax.rsqrt(norm_var[0] + 1e-6)
    mean16 = jnp.broadcast_to(norm_mean[0], (L,))
    scale16 = jnp.broadcast_to(scale, (L,))
    utab_t = user_table.T
    ttab_p = jnp.pad(ts_table, ((0, TSROWS - ts_table.shape[0]), (0, 0))).T
    return _run(user_id.astype(jnp.int32), timestamp, utab_t, ttab_p,
                buckets, mean16, scale16)
